# Initial kernel scaffold; baseline (speedup 1.0000x reference)
import jax
import jax.numpy as jnp
from jax.experimental import pallas as pl


def kernel(instance_feature, anchor, anchor_embed, feature_maps, W_proj, b_proj):
    raise NotImplementedError("write your pallas kernel here")


# speedup vs baseline: 1.0815x; 1.0815x over previous
import functools

import jax
import jax.numpy as jnp
from jax import lax
from jax.experimental import pallas as pl
from jax.experimental.pallas import tpu as pltpu
from jax.experimental.pallas import tpu_sc as plsc

BS, NA, D = 2, 900, 256
C, H, W = 256, 200, 200
HW = H * W
L = 16
NC, NS = 2, 16
NW = NC * NS
APW = 64
NPAD = NW * APW

XMIN, XMAX = -80.0, 120.0
YMIN, YMAX = -40.0, 40.0

_mesh = plsc.VectorSubcoreMesh(
    core_axis_name="c", subcore_axis_name="s", num_cores=NC, num_subcores=NS)


def _floor_i(x):
    t = x.astype(jnp.int32)
    tf = t.astype(jnp.float32)
    t = t - jnp.where(tf > x, 1, 0)
    return t, t.astype(jnp.float32)


@functools.partial(
    pl.kernel,
    out_type=jax.ShapeDtypeStruct((NPAD, C), jnp.float32),
    mesh=_mesh,
    scratch_types=[
        pltpu.VMEM((APW,), jnp.float32),
        pltpu.VMEM((APW,), jnp.float32),
        pltpu.VMEM((4 * APW,), jnp.int32),
        pltpu.VMEM((4 * APW,), jnp.float32),
        pltpu.VMEM((4 * APW, C), jnp.float32),
        pltpu.VMEM((APW, C), jnp.float32),
        pltpu.SemaphoreType.DMA,
    ],
    compiler_params=pltpu.CompilerParams(needs_layout_passes=False),
)
def _sc_gather(ax_hbm, ay_hbm, bev_hbm, out_hbm,
               ax_v, ay_v, idx_v, w_v, rows_v, feats_v, sem):
    wid = lax.axis_index("s") * NC + lax.axis_index("c")
    base = wid * APW
    pltpu.sync_copy(ax_hbm.at[pl.ds(base, APW)], ax_v)
    pltpu.sync_copy(ay_hbm.at[pl.ds(base, APW)], ay_v)

    for g in range(APW // L):
        x = ax_v[pl.ds(g * L, L)]
        y = ay_v[pl.ds(g * L, L)]
        gx = (y - YMIN) / (YMAX - YMIN + 1e-06) * 2.0 - 1.0
        gy = (x - XMIN) / (XMAX - XMIN + 1e-06) * 2.0 - 1.0
        ix = (gx + 1.0) * 0.5 * (W - 1)
        iy = (gy + 1.0) * 0.5 * (H - 1)
        ix = jnp.clip(ix, -4.0, W + 4.0)
        iy = jnp.clip(iy, -4.0, H + 4.0)
        x0, x0f = _floor_i(ix)
        y0, y0f = _floor_i(iy)
        dx0 = ix - x0f
        dx1 = (x0f + 1.0) - ix
        dy0 = iy - y0f
        dy1 = (y0f + 1.0) - iy
        vx0 = (x0 >= 0) & (x0 < W)
        vx1 = (x0 >= -1) & (x0 < W - 1)
        vy0 = (y0 >= 0) & (y0 < H)
        vy1 = (y0 >= -1) & (y0 < H - 1)
        xc0 = jnp.clip(x0, 0, W - 1)
        xc1 = jnp.clip(x0 + 1, 0, W - 1)
        yo0 = jnp.clip(y0, 0, H - 1) * W
        yo1 = jnp.clip(y0 + 1, 0, H - 1) * W
        aid = base + g * L + lax.iota(jnp.int32, 16)
        boff = jnp.where(aid >= NA, HW, 0)
        corners = (
            (yo0, xc0, vy0 & vx0, dx1 * dy1),
            (yo1, xc0, vy1 & vx0, dx1 * dy0),
            (yo0, xc1, vy0 & vx1, dx0 * dy1),
            (yo1, xc1, vy1 & vx1, dx0 * dy0),
        )
        for k, (yo, xc, valid, wgt) in enumerate(corners):
            p = g * (4 * L) + k * L
            idx_v[pl.ds(p, L)] = boff + yo + xc
            w_v[pl.ds(p, L)] = jnp.where(valid, wgt, 0.0)

    pltpu.async_copy(bev_hbm.at[idx_v], rows_v, sem).wait()

    def body(a, carry):
        g = a // L
        i = a - g * L
        rbase = g * (4 * L) + i
        accs = [None] * (C // L)
        for k in range(4):
            r = rbase + k * L
            wv = plsc.load_gather(w_v, [jnp.full((L,), r, dtype=jnp.int32)])
            for j in range(C // L):
                chunk = rows_v[r, pl.ds(j * L, L)]
                accs[j] = wv * chunk if k == 0 else accs[j] + wv * chunk
        for j in range(C // L):
            feats_v[a, pl.ds(j * L, L)] = accs[j]
        return carry

    lax.fori_loop(0, APW, body, 0)
    pltpu.sync_copy(feats_v, out_hbm.at[pl.ds(base, APW)])


def _mm_body(f_ref, w_ref, b_ref, inst_ref, o_ref):
    o_ref[...] = (
        lax.dot_general(f_ref[...], w_ref[...], (((1,), (1,)), ((), ())),
                        preferred_element_type=jnp.float32)
        + b_ref[...] + inst_ref[...])


def _tc_proj(feats, w_proj, b2, inst):
    return pl.pallas_call(
        _mm_body,
        out_shape=jax.ShapeDtypeStruct((BS * NA, D), jnp.float32),
    )(feats, w_proj, b2, inst)


def kernel(instance_feature, anchor, anchor_embed, feature_maps, W_proj, b_proj):
    ax = jnp.pad(anchor[..., 0].reshape(-1), (0, NPAD - BS * NA))
    ay = jnp.pad(anchor[..., 1].reshape(-1), (0, NPAD - BS * NA))
    bev = feature_maps.transpose(0, 2, 3, 1).reshape(BS * HW, C)
    feats = _sc_gather(ax, ay, bev)
    out = _tc_proj(feats[:BS * NA], W_proj, b_proj.reshape(1, D),
                   instance_feature.reshape(BS * NA, D))
    return out.reshape(BS, NA, D)
